# Initial kernel scaffold; baseline (speedup 1.0000x reference)
import jax
import jax.numpy as jnp
from jax.experimental import pallas as pl


def kernel(x, edge_index, dropout, Wh, bh, ah, Wo, bo, ao):
    raise NotImplementedError("write your pallas kernel here")


# speedup vs baseline: 20.6799x; 20.6799x over previous
import functools

import jax
import jax.numpy as jnp
from jax import lax
from jax.experimental import pallas as pl
from jax.experimental.pallas import tpu as pltpu
from jax.experimental.pallas import tpu_sc as plsc

N = 10000
NP = 10240
NPAD = NP - N
E = 320000
NC = 2
NS = 16
NW = NC * NS
EW = E // NW
CK = 128
NCH = 80
EWP = NCH * CK
IN_F = 128
HID = 32
HEADS = 4
F1 = HID * HEADS
F2 = 64
RPT = NP // NS
BLK = 1024
GRID = NP // BLK


def _lin1_body(x_ref, wt_ref, b_ref, a1_ref, a2_ref, x1_ref, s1t_ref, s2t_ref):
    x1 = jnp.dot(x_ref[...], wt_ref[...], preferred_element_type=jnp.float32)
    x1 = x1 + b_ref[...]
    x1_ref[...] = x1
    dn = (((0,), (1,)), ((), ()))
    s1t_ref[...] = lax.dot_general(a1_ref[...], x1, dn,
                                   preferred_element_type=jnp.float32)
    s2t_ref[...] = lax.dot_general(a2_ref[...], x1, dn,
                                   preferred_element_type=jnp.float32)


def _lin1(xp, wcatT, bcat, a1, a2):
    return pl.pallas_call(
        _lin1_body,
        grid=(GRID,),
        in_specs=[
            pl.BlockSpec((BLK, IN_F), lambda i: (i, 0)),
            pl.BlockSpec((IN_F, F1), lambda i: (0, 0)),
            pl.BlockSpec((1, F1), lambda i: (0, 0)),
            pl.BlockSpec((F1, HEADS), lambda i: (0, 0)),
            pl.BlockSpec((F1, HEADS), lambda i: (0, 0)),
        ],
        out_specs=[
            pl.BlockSpec((BLK, F1), lambda i: (i, 0)),
            pl.BlockSpec((HEADS, BLK), lambda i: (0, i)),
            pl.BlockSpec((HEADS, BLK), lambda i: (0, i)),
        ],
        out_shape=[
            jax.ShapeDtypeStruct((NP, F1), jnp.float32),
            jax.ShapeDtypeStruct((HEADS, NP), jnp.float32),
            jax.ShapeDtypeStruct((HEADS, NP), jnp.float32),
        ],
    )(xp, wcatT, bcat, a1, a2)


def _comb_body(d_ref, x_ref, erep_ref, xs_ref):
    dsum = d_ref[0] + d_ref[1]
    dinv = 1.0 / dsum
    mult = lax.dot_general(dinv, erep_ref[...], (((0,), (0,)), ((), ())),
                           preferred_element_type=jnp.float32)
    xs_ref[...] = x_ref[...] * mult


def _comb(denom, xarr, erep):
    nh, f = erep.shape
    return pl.pallas_call(
        _comb_body,
        grid=(GRID,),
        in_specs=[
            pl.BlockSpec((2, nh, BLK), lambda i: (0, 0, i)),
            pl.BlockSpec((BLK, f), lambda i: (i, 0)),
            pl.BlockSpec((nh, f), lambda i: (0, 0)),
        ],
        out_specs=pl.BlockSpec((BLK, f), lambda i: (i, 0)),
        out_shape=jax.ShapeDtypeStruct((NP, f), jnp.float32),
    )(denom, xarr, erep)


def _lin2_body(p_ref, wt_ref, b_ref, a1_ref, a2_ref, x2_ref, s1t_ref, s2t_ref):
    i = pl.program_id(0)
    h = p_ref[0] + p_ref[1]
    gid = i * BLK + lax.broadcasted_iota(jnp.int32, (BLK, 1), 0)
    h = jnp.where(gid < N, h, 0.0)
    x2 = jnp.dot(h, wt_ref[...], preferred_element_type=jnp.float32)
    x2 = x2 + b_ref[...]
    x2_ref[...] = x2
    dn = (((0,), (1,)), ((), ()))
    s1t_ref[...] = lax.dot_general(a1_ref[...], x2, dn,
                                   preferred_element_type=jnp.float32)
    s2t_ref[...] = lax.dot_general(a2_ref[...], x2, dn,
                                   preferred_element_type=jnp.float32)


def _lin2(outp1, woT, bo2, a1o, a2o):
    return pl.pallas_call(
        _lin2_body,
        grid=(GRID,),
        in_specs=[
            pl.BlockSpec((2, BLK, F1), lambda i: (0, i, 0)),
            pl.BlockSpec((F1, F1), lambda i: (0, 0)),
            pl.BlockSpec((1, F1), lambda i: (0, 0)),
            pl.BlockSpec((F1, 1), lambda i: (0, 0)),
            pl.BlockSpec((F1, 1), lambda i: (0, 0)),
        ],
        out_specs=[
            pl.BlockSpec((BLK, F1), lambda i: (i, 0)),
            pl.BlockSpec((1, BLK), lambda i: (0, i)),
            pl.BlockSpec((1, BLK), lambda i: (0, i)),
        ],
        out_shape=[
            jax.ShapeDtypeStruct((NP, F1), jnp.float32),
            jax.ShapeDtypeStruct((1, NP), jnp.float32),
            jax.ShapeDtypeStruct((1, NP), jnp.float32),
        ],
    )(outp1, woT, bo2, a1o, a2o)


def _final_body(p_ref, o_ref):
    s = p_ref[0] + p_ref[1]
    s = s[:, :F2]
    e = jnp.where(s > 0, s, jnp.exp(s) - 1.0)
    m = jnp.max(e, axis=1, keepdims=True)
    z = jnp.exp(e - m)
    o_ref[...] = z / jnp.sum(z, axis=1, keepdims=True)


def _final(outp2):
    return pl.pallas_call(
        _final_body,
        grid=(GRID,),
        in_specs=[pl.BlockSpec((2, BLK, F1), lambda i: (0, i, 0))],
        out_specs=pl.BlockSpec((BLK, F2), lambda i: (i, 0)),
        out_shape=jax.ShapeDtypeStruct((NP, F2), jnp.float32),
    )(outp2)


_MESH = plsc.VectorSubcoreMesh(core_axis_name="c", subcore_axis_name="s",
                               num_cores=NC, num_subcores=NS)


def _make_edge_w(nh):
    @functools.partial(
        pl.kernel,
        out_type=(
            jax.ShapeDtypeStruct((NC, nh, NP), jnp.float32),
            jax.ShapeDtypeStruct((NW, NCH, CK * nh), jnp.float32),
        ),
        mesh=_MESH,
        scratch_types=[
            pltpu.VMEM((nh * NP,), jnp.float32),
            pltpu.VMEM((nh * NP,), jnp.float32),
            pltpu.VMEM((NCH, CK), jnp.int32),
            pltpu.VMEM((NCH, CK), jnp.int32),
            pltpu.VMEM((nh, CK), jnp.float32),
            pltpu.VMEM((CK * nh,), jnp.float32),
            [pltpu.VMEM_SHARED((NP,), jnp.float32)] * nh,
        ],
        compiler_params=pltpu.CompilerParams(needs_layout_passes=False),
    )
    def k(s1_hbm, s2_hbm, row_hbm, col_hbm, denom_hbm, w_hbm,
          s1_v, s2_v, row_v, col_v, wt_v, w_v, accs):
        cid = lax.axis_index("c")
        sid = lax.axis_index("s")
        wid = sid * NC + cid
        zf = jnp.zeros((16,), jnp.float32)
        for h in range(nh):
            for g in range(8):
                wt_v[h, pl.ds(g * 16, 16)] = zf
        for h in range(nh):
            for t in range(RPT // CK):
                pltpu.sync_copy(
                    wt_v.at[h],
                    accs[h].at[pl.ds(sid * RPT + t * CK, CK)])
        plsc.subcore_barrier()
        for h in range(nh):
            pltpu.sync_copy(s1_hbm.at[h], s1_v.at[pl.ds(h * NP, NP)])
            pltpu.sync_copy(s2_hbm.at[h], s2_v.at[pl.ds(h * NP, NP)])
        pltpu.sync_copy(row_hbm.at[wid], row_v)
        pltpu.sync_copy(col_hbm.at[wid], col_v)

        def chunk(j, carry):
            for g in range(8):
                kv = g * 16 + lax.iota(jnp.int32, 16)
                ridx = row_v[j, pl.ds(g * 16, 16)]
                cidx = col_v[j, pl.ds(g * 16, 16)]
                for h in range(nh):
                    s1 = plsc.load_gather(s1_v, [ridx + (h * NP)])
                    s2 = plsc.load_gather(s2_v, [cidx + (h * NP)])
                    t = s1 + s2
                    e = jnp.where(t > 0, t, t * 0.01)
                    w = jnp.exp(e)
                    wt_v[h, pl.ds(g * 16, 16)] = w
                    plsc.store_scatter(w_v, [kv * nh + h], w)
            for h in range(nh):
                pltpu.sync_copy(wt_v.at[h], accs[h].at[row_v.at[j]],
                                add=True)
            pltpu.sync_copy(w_v, w_hbm.at[wid, j])
            return carry

        lax.fori_loop(0, NCH, chunk, 0)
        plsc.subcore_barrier()
        for h in range(nh):
            pltpu.sync_copy(accs[h].at[pl.ds(sid * RPT, RPT)],
                            denom_hbm.at[cid, h, pl.ds(sid * RPT, RPT)])

    return k


def _make_edge_agg(nh, F):
    nblk = F // 16
    fph = F // nh

    @functools.partial(
        pl.kernel,
        out_type=jax.ShapeDtypeStruct((NC, NP, F), jnp.float32),
        mesh=_MESH,
        scratch_types=[
            pltpu.VMEM((NCH, CK), jnp.int32),
            pltpu.VMEM((NCH, CK), jnp.int32),
            pltpu.VMEM((CK * nh + 16,), jnp.float32),
            pltpu.VMEM((CK, F), jnp.float32),
            pltpu.VMEM_SHARED((NP, F), jnp.float32),
        ],
        compiler_params=pltpu.CompilerParams(needs_layout_passes=False),
    )
    def k(xs_hbm, row_hbm, col_hbm, w_hbm, out_hbm,
          row_v, col_v, w_v, msg_v, acc_sh):
        cid = lax.axis_index("c")
        sid = lax.axis_index("s")
        wid = sid * NC + cid
        zf = jnp.zeros((16,), jnp.float32)

        def zrow(i, carry):
            for b in range(nblk):
                msg_v[i, pl.ds(b * 16, 16)] = zf
            return carry

        lax.fori_loop(0, CK, zrow, 0)
        for t in range(RPT // CK):
            pltpu.sync_copy(msg_v, acc_sh.at[pl.ds(sid * RPT + t * CK, CK)])
        plsc.subcore_barrier()
        pltpu.sync_copy(row_hbm.at[wid], row_v)
        pltpu.sync_copy(col_hbm.at[wid], col_v)

        def chunk(j, carry):
            pltpu.sync_copy(w_hbm.at[wid, j], w_v.at[pl.ds(0, CK * nh)])
            pltpu.sync_copy(xs_hbm.at[row_v.at[j]], msg_v)

            def edge(kk, c2):
                wvec = w_v[pl.ds(kk * nh, 16)]
                for b in range(nblk):
                    h = (b * 16) // fph
                    v = msg_v[kk, pl.ds(b * 16, 16)]
                    msg_v[kk, pl.ds(b * 16, 16)] = v * wvec[h]
                return c2

            lax.fori_loop(0, CK, edge, 0)
            pltpu.sync_copy(msg_v, acc_sh.at[col_v.at[j]], add=True)
            return carry

        lax.fori_loop(0, NCH, chunk, 0)
        plsc.subcore_barrier()
        pltpu.sync_copy(acc_sh.at[pl.ds(sid * RPT, RPT)],
                        out_hbm.at[cid, pl.ds(sid * RPT, RPT)])

    return k


_edge_w4 = _make_edge_w(4)
_edge_w1 = _make_edge_w(1)
_edge_agg4 = _make_edge_agg(4, F1)
_edge_agg1 = _make_edge_agg(1, F1)


def kernel(x, edge_index, dropout, Wh, bh, ah, Wo, bo, ao):
    f32 = jnp.float32
    x = x.astype(f32)
    row = edge_index[0].astype(jnp.int32)
    col = edge_index[1].astype(jnp.int32)

    xp = jnp.pad(x, ((0, NPAD), (0, 0)))
    wcatT = Wh.reshape(F1, IN_F).T
    bcat = bh.reshape(1, F1)
    eye = jnp.eye(HEADS, dtype=f32)
    a1 = (eye[:, None, :] * ah[:, :HID, 0][:, :, None]).reshape(F1, HEADS)
    a2 = (eye[:, None, :] * ah[:, HID:, 0][:, :, None]).reshape(F1, HEADS)
    erep = jnp.repeat(eye, HID, axis=1)
    erep2 = jnp.ones((1, F1), f32)
    woT = jnp.pad(Wo.T, ((0, 0), (0, F1 - F2)))
    bo2 = jnp.pad(bo.reshape(1, F2), ((0, 0), (0, F1 - F2)))
    a1o = jnp.pad(ao[:F2], ((0, F1 - F2), (0, 0)))
    a2o = jnp.pad(ao[F2:], ((0, F1 - F2), (0, 0)))

    dummy = jnp.broadcast_to(N + jnp.arange(NPAD, dtype=jnp.int32),
                             (NW, NPAD))
    rowp = jnp.concatenate([row.reshape(NW, EW), dummy], axis=1)
    rowp = rowp.reshape(NW, NCH, CK)
    colp = jnp.concatenate([col.reshape(NW, EW), dummy], axis=1)
    colp = colp.reshape(NW, NCH, CK)

    x1, s1t, s2t = _lin1(xp, wcatT, bcat, a1, a2)
    denom1, w1 = _edge_w4(s1t, s2t, rowp, colp)
    xs1 = _comb(denom1, x1, erep)
    outp1 = _edge_agg4(xs1, rowp, colp, w1)

    x2, s1ot, s2ot = _lin2(outp1, woT, bo2, a1o, a2o)
    denom2, w2 = _edge_w1(s1ot, s2ot, rowp, colp)
    xs2 = _comb(denom2, x2, erep2)
    outp2 = _edge_agg1(xs2, rowp, colp, w2)

    out = _final(outp2)
    return out[:N]


# speedup vs baseline: 29.7506x; 1.4386x over previous
import functools

import jax
import jax.numpy as jnp
from jax import lax
from jax.experimental import pallas as pl
from jax.experimental.pallas import tpu as pltpu
from jax.experimental.pallas import tpu_sc as plsc

N = 10000
NP = 10240
NPAD = NP - N
E = 320000
NC = 2
NS = 16
NW = NC * NS
EW = E // NW
CK = 128
EWP = 10368
NCHA = EWP // CK
CKB = 128
NCHB = EWP // CKB
IN_F = 128
HID = 32
HEADS = 4
F1 = HID * HEADS
F2 = 64
RPT = NP // NS
BLK = 1024
GRID = NP // BLK


def _lin1_body(x_ref, wt_ref, b_ref, a1_ref, a2_ref, x1_ref, s1t_ref, s2t_ref):
    x1 = jnp.dot(x_ref[...], wt_ref[...], preferred_element_type=jnp.float32)
    x1 = x1 + b_ref[...]
    x1_ref[...] = x1
    dn = (((0,), (1,)), ((), ()))
    s1t_ref[...] = lax.dot_general(a1_ref[...], x1, dn,
                                   preferred_element_type=jnp.float32)
    s2t_ref[...] = lax.dot_general(a2_ref[...], x1, dn,
                                   preferred_element_type=jnp.float32)


def _lin1(xp, wcatT, bcat, a1, a2):
    return pl.pallas_call(
        _lin1_body,
        grid=(GRID,),
        in_specs=[
            pl.BlockSpec((BLK, IN_F), lambda i: (i, 0)),
            pl.BlockSpec((IN_F, F1), lambda i: (0, 0)),
            pl.BlockSpec((1, F1), lambda i: (0, 0)),
            pl.BlockSpec((F1, HEADS), lambda i: (0, 0)),
            pl.BlockSpec((F1, HEADS), lambda i: (0, 0)),
        ],
        out_specs=[
            pl.BlockSpec((BLK, F1), lambda i: (i, 0)),
            pl.BlockSpec((HEADS, BLK), lambda i: (0, i)),
            pl.BlockSpec((HEADS, BLK), lambda i: (0, i)),
        ],
        out_shape=[
            jax.ShapeDtypeStruct((NP, F1), jnp.float32),
            jax.ShapeDtypeStruct((HEADS, NP), jnp.float32),
            jax.ShapeDtypeStruct((HEADS, NP), jnp.float32),
        ],
    )(xp, wcatT, bcat, a1, a2)


def _comb_body(d_ref, x_ref, erep_ref, xs_ref):
    dsum = d_ref[0] + d_ref[1]
    dinv = 1.0 / dsum
    mult = lax.dot_general(dinv, erep_ref[...], (((0,), (0,)), ((), ())),
                           preferred_element_type=jnp.float32)
    xs_ref[...] = x_ref[...] * mult


def _comb(denom, xarr, erep):
    nh, f = erep.shape
    return pl.pallas_call(
        _comb_body,
        grid=(GRID,),
        in_specs=[
            pl.BlockSpec((2, nh, BLK), lambda i: (0, 0, i)),
            pl.BlockSpec((BLK, f), lambda i: (i, 0)),
            pl.BlockSpec((nh, f), lambda i: (0, 0)),
        ],
        out_specs=pl.BlockSpec((BLK, f), lambda i: (i, 0)),
        out_shape=jax.ShapeDtypeStruct((NP, f), jnp.float32),
    )(denom, xarr, erep)


def _lin2_body(p_ref, wt_ref, b_ref, a1_ref, a2_ref, x2_ref, s1t_ref, s2t_ref):
    i = pl.program_id(0)
    h = p_ref[0] + p_ref[1]
    gid = i * BLK + lax.broadcasted_iota(jnp.int32, (BLK, 1), 0)
    h = jnp.where(gid < N, h, 0.0)
    x2 = jnp.dot(h, wt_ref[...], preferred_element_type=jnp.float32)
    x2 = x2 + b_ref[...]
    x2_ref[...] = x2
    dn = (((0,), (1,)), ((), ()))
    s1t_ref[...] = lax.dot_general(a1_ref[...], x2, dn,
                                   preferred_element_type=jnp.float32)
    s2t_ref[...] = lax.dot_general(a2_ref[...], x2, dn,
                                   preferred_element_type=jnp.float32)


def _lin2(outp1, woT, bo2, a1o, a2o):
    return pl.pallas_call(
        _lin2_body,
        grid=(GRID,),
        in_specs=[
            pl.BlockSpec((2, BLK, F1), lambda i: (0, i, 0)),
            pl.BlockSpec((F1, F1), lambda i: (0, 0)),
            pl.BlockSpec((1, F1), lambda i: (0, 0)),
            pl.BlockSpec((F1, 1), lambda i: (0, 0)),
            pl.BlockSpec((F1, 1), lambda i: (0, 0)),
        ],
        out_specs=[
            pl.BlockSpec((BLK, F1), lambda i: (i, 0)),
            pl.BlockSpec((1, BLK), lambda i: (0, i)),
            pl.BlockSpec((1, BLK), lambda i: (0, i)),
        ],
        out_shape=[
            jax.ShapeDtypeStruct((NP, F1), jnp.float32),
            jax.ShapeDtypeStruct((1, NP), jnp.float32),
            jax.ShapeDtypeStruct((1, NP), jnp.float32),
        ],
    )(outp1, woT, bo2, a1o, a2o)


def _final_body(p_ref, o_ref):
    s = p_ref[0] + p_ref[1]
    s = s[:, :F2]
    e = jnp.where(s > 0, s, jnp.exp(s) - 1.0)
    m = jnp.max(e, axis=1, keepdims=True)
    z = jnp.exp(e - m)
    o_ref[...] = z / jnp.sum(z, axis=1, keepdims=True)


def _final(outp2):
    return pl.pallas_call(
        _final_body,
        grid=(GRID,),
        in_specs=[pl.BlockSpec((2, BLK, F1), lambda i: (0, i, 0))],
        out_specs=pl.BlockSpec((BLK, F2), lambda i: (i, 0)),
        out_shape=jax.ShapeDtypeStruct((NP, F2), jnp.float32),
    )(outp2)


_MESH = plsc.VectorSubcoreMesh(core_axis_name="c", subcore_axis_name="s",
                               num_cores=NC, num_subcores=NS)


def _make_edge_w(nh):
    @functools.partial(
        pl.kernel,
        out_type=(
            jax.ShapeDtypeStruct((NC, nh, NP), jnp.float32),
            jax.ShapeDtypeStruct((NW, NCHA, CK * nh), jnp.float32),
        ),
        mesh=_MESH,
        scratch_types=[
            pltpu.VMEM((nh * NP,), jnp.float32),
            pltpu.VMEM((nh * NP,), jnp.float32),
            pltpu.VMEM((NCHA, CK), jnp.int32),
            pltpu.VMEM((NCHA, CK), jnp.int32),
            pltpu.VMEM((nh, CK), jnp.float32),
            pltpu.VMEM((CK * nh,), jnp.float32),
            [pltpu.VMEM_SHARED((NP,), jnp.float32)] * nh,
        ],
        compiler_params=pltpu.CompilerParams(needs_layout_passes=False),
    )
    def k(s1_hbm, s2_hbm, row_hbm, col_hbm, denom_hbm, w_hbm,
          s1_v, s2_v, row_v, col_v, wt_v, w_v, accs):
        cid = lax.axis_index("c")
        sid = lax.axis_index("s")
        wid = sid * NC + cid
        zf = jnp.zeros((16,), jnp.float32)
        for h in range(nh):
            for g in range(8):
                wt_v[h, pl.ds(g * 16, 16)] = zf
        for h in range(nh):
            for t in range(RPT // CK):
                pltpu.sync_copy(
                    wt_v.at[h],
                    accs[h].at[pl.ds(sid * RPT + t * CK, CK)])
        plsc.subcore_barrier()
        for h in range(nh):
            pltpu.sync_copy(s1_hbm.at[h], s1_v.at[pl.ds(h * NP, NP)])
            pltpu.sync_copy(s2_hbm.at[h], s2_v.at[pl.ds(h * NP, NP)])
        pltpu.sync_copy(row_hbm.at[wid], row_v)
        pltpu.sync_copy(col_hbm.at[wid], col_v)

        def chunk(j, carry):
            for g in range(8):
                kv = g * 16 + lax.iota(jnp.int32, 16)
                ridx = row_v[j, pl.ds(g * 16, 16)]
                cidx = col_v[j, pl.ds(g * 16, 16)]
                for h in range(nh):
                    s1 = plsc.load_gather(s1_v, [ridx + (h * NP)])
                    s2 = plsc.load_gather(s2_v, [cidx + (h * NP)])
                    t = s1 + s2
                    e = jnp.where(t > 0, t, t * 0.01)
                    w = jnp.exp(e)
                    wt_v[h, pl.ds(g * 16, 16)] = w
                    plsc.store_scatter(w_v, [kv * nh + h], w)
            for h in range(nh):
                pltpu.sync_copy(wt_v.at[h], accs[h].at[row_v.at[j]],
                                add=True)
            pltpu.sync_copy(w_v, w_hbm.at[wid, j])
            return carry

        lax.fori_loop(0, NCHA, chunk, 0)
        plsc.subcore_barrier()
        for h in range(nh):
            pltpu.sync_copy(accs[h].at[pl.ds(sid * RPT, RPT)],
                            denom_hbm.at[cid, h, pl.ds(sid * RPT, RPT)])

    return k


def _make_edge_agg(nh, F):
    nblk = F // 16
    fph = F // nh

    @functools.partial(
        pl.kernel,
        out_type=jax.ShapeDtypeStruct((NC, NP, F), jnp.float32),
        mesh=_MESH,
        scratch_types=[
            [pltpu.VMEM((1, CKB), jnp.int32)] * 2,
            [pltpu.VMEM((1, CKB), jnp.int32)] * 2,
            [pltpu.VMEM((1, CKB * nh + 16), jnp.float32)] * 2,
            [pltpu.VMEM((CKB, F), jnp.float32)] * 2,
            [pltpu.SemaphoreType.DMA] * 2,
            [pltpu.SemaphoreType.DMA] * 2,
            [pltpu.SemaphoreType.DMA] * 2,
            [pltpu.SemaphoreType.DMA] * 2,
            pltpu.VMEM_SHARED((NP, F), jnp.float32),
        ],
        compiler_params=pltpu.CompilerParams(needs_layout_passes=False),
    )
    def k(xs_hbm, row_hbm, col_hbm, w_hbm, out_hbm,
          ribufs, cibufs, wbufs, msgs, risems, cisems, gsems, wsems,
          acc_sh):
        cid = lax.axis_index("c")
        sid = lax.axis_index("s")
        wid = sid * NC + cid
        zf = jnp.zeros((16,), jnp.float32)

        def zrow(i, carry):
            for b in range(nblk):
                msgs[0][i, pl.ds(b * 16, 16)] = zf
            return carry

        lax.fori_loop(0, CKB, zrow, 0)
        for t in range(RPT // CKB):
            pltpu.sync_copy(msgs[0],
                            acc_sh.at[pl.ds(sid * RPT + t * CKB, CKB)])
        plsc.subcore_barrier()

        def ristart(j, b):
            pltpu.async_copy(row_hbm.at[wid, j], ribufs[b], risems[b])

        def riwait(j, b):
            pltpu.make_async_copy(row_hbm.at[wid, j], ribufs[b],
                                  risems[b]).wait()

        def cistart(j, b):
            pltpu.async_copy(col_hbm.at[wid, j], cibufs[b], cisems[b])

        def ciwait(j, b):
            pltpu.make_async_copy(col_hbm.at[wid, j], cibufs[b],
                                  cisems[b]).wait()

        def gstart(j, b):
            pltpu.async_copy(w_hbm.at[wid, j],
                             wbufs[b].at[:, pl.ds(0, CKB * nh)], wsems[b])
            pltpu.async_copy(xs_hbm.at[ribufs[b].at[0]], msgs[b], gsems[b])

        def gwait(j, b):
            pltpu.make_async_copy(w_hbm.at[wid, j],
                                  wbufs[b].at[:, pl.ds(0, CKB * nh)],
                                  wsems[b]).wait()
            pltpu.make_async_copy(xs_hbm.at[ribufs[b].at[0]], msgs[b],
                                  gsems[b]).wait()

        def compute(j, b):
            mb = msgs[b]
            wb = wbufs[b]

            def edge(kk, c2):
                wvec = wb[0, pl.ds(kk * nh, 16)]
                for bb in range(nblk):
                    h = (bb * 16) // fph
                    v = mb[kk, pl.ds(bb * 16, 16)]
                    mb[kk, pl.ds(bb * 16, 16)] = v * wvec[h]
                return c2

            lax.fori_loop(0, CKB, edge, 0)

        pltpu.sync_copy(row_hbm.at[wid, 0], ribufs[0])
        pltpu.sync_copy(col_hbm.at[wid, 0], cibufs[0])
        ristart(1, 1)
        cistart(1, 1)
        gstart(0, 0)
        half = NCHB // 2

        def outer(i, carry):
            for b in range(2):
                j = i * 2 + b
                riwait(j + 1, 1 - b)
                gstart(j + 1, 1 - b)
                gwait(j, b)

                @pl.when(j + 2 < NCHB)
                def _():
                    ristart(j + 2, b)
                compute(j, b)
                if b == 0:
                    @pl.when(i > 0)
                    def _():
                        ciwait(j, b)
                else:
                    ciwait(j, b)
                pltpu.sync_copy(msgs[b], acc_sh.at[cibufs[b].at[0]],
                                add=True)

                @pl.when(j + 2 < NCHB)
                def _():
                    cistart(j + 2, b)
            return carry

        lax.fori_loop(0, half, outer, 0)
        jl = NCHB - 1
        gwait(jl, 0)
        compute(jl, 0)
        ciwait(jl, 0)
        pltpu.sync_copy(msgs[0], acc_sh.at[cibufs[0].at[0]], add=True)
        plsc.subcore_barrier()
        pltpu.sync_copy(acc_sh.at[pl.ds(sid * RPT, RPT)],
                        out_hbm.at[cid, pl.ds(sid * RPT, RPT)])

    return k


_edge_w4 = _make_edge_w(4)
_edge_w1 = _make_edge_w(1)
_edge_agg4 = _make_edge_agg(4, F1)
_edge_agg1 = _make_edge_agg(1, F1)


def kernel(x, edge_index, dropout, Wh, bh, ah, Wo, bo, ao):
    f32 = jnp.float32
    x = x.astype(f32)
    row = edge_index[0].astype(jnp.int32)
    col = edge_index[1].astype(jnp.int32)

    xp = jnp.pad(x, ((0, NPAD), (0, 0)))
    wcatT = Wh.reshape(F1, IN_F).T
    bcat = bh.reshape(1, F1)
    eye = jnp.eye(HEADS, dtype=f32)
    a1 = (eye[:, None, :] * ah[:, :HID, 0][:, :, None]).reshape(F1, HEADS)
    a2 = (eye[:, None, :] * ah[:, HID:, 0][:, :, None]).reshape(F1, HEADS)
    erep = jnp.repeat(eye, HID, axis=1)
    erep2 = jnp.ones((1, F1), f32)
    woT = jnp.pad(Wo.T, ((0, 0), (0, F1 - F2)))
    bo2 = jnp.pad(bo.reshape(1, F2), ((0, 0), (0, F1 - F2)))
    a1o = jnp.pad(ao[:F2], ((0, F1 - F2), (0, 0)))
    a2o = jnp.pad(ao[F2:], ((0, F1 - F2), (0, 0)))

    dummy = jnp.broadcast_to(
        N + jnp.arange(EWP - EW, dtype=jnp.int32) % NPAD, (NW, EWP - EW))
    rowf = jnp.concatenate([row.reshape(NW, EW), dummy], axis=1)
    colf = jnp.concatenate([col.reshape(NW, EW), dummy], axis=1)
    rowp = rowf.reshape(NW, NCHA, CK)
    colp = colf.reshape(NW, NCHA, CK)
    rowb = rowf.reshape(NW, NCHB, 1, CKB)
    colb = colf.reshape(NW, NCHB, 1, CKB)

    x1, s1t, s2t = _lin1(xp, wcatT, bcat, a1, a2)
    denom1, w1 = _edge_w4(s1t, s2t, rowp, colp)
    xs1 = _comb(denom1, x1, erep)
    outp1 = _edge_agg4(xs1, rowb, colb,
                       w1.reshape(NW, NCHB, 1, CKB * HEADS))

    x2, s1ot, s2ot = _lin2(outp1, woT, bo2, a1o, a2o)
    denom2, w2 = _edge_w1(s1ot, s2ot, rowp, colp)
    xs2 = _comb(denom2, x2, erep2)
    outp2 = _edge_agg1(xs2, rowb, colb, w2.reshape(NW, NCHB, 1, CKB))

    out = _final(outp2)
    return out[:N]


# speedup vs baseline: 30.4255x; 1.0227x over previous
import functools

import jax
import jax.numpy as jnp
from jax import lax
from jax.experimental import pallas as pl
from jax.experimental.pallas import tpu as pltpu
from jax.experimental.pallas import tpu_sc as plsc

N = 10000
NP = 10240
NPAD = NP - N
E = 320000
NC = 2
NS = 16
NW = NC * NS
EW = E // NW
CK = 128
EWP = 10368
NCHA = EWP // CK
CKB = 128
NCHB = EWP // CKB
IN_F = 128
HID = 32
HEADS = 4
F1 = HID * HEADS
F2 = 64
RPT = NP // NS
BLK = 1024
GRID = NP // BLK


def _lin1_body(x_ref, wt_ref, b_ref, a1_ref, a2_ref, x1_ref, s1t_ref, s2t_ref):
    x1 = jnp.dot(x_ref[...], wt_ref[...], preferred_element_type=jnp.float32)
    x1 = x1 + b_ref[...]
    x1_ref[...] = x1
    dn = (((0,), (1,)), ((), ()))
    s1t_ref[...] = lax.dot_general(a1_ref[...], x1, dn,
                                   preferred_element_type=jnp.float32)
    s2t_ref[...] = lax.dot_general(a2_ref[...], x1, dn,
                                   preferred_element_type=jnp.float32)


def _lin1(xp, wcatT, bcat, a1, a2):
    return pl.pallas_call(
        _lin1_body,
        grid=(GRID,),
        in_specs=[
            pl.BlockSpec((BLK, IN_F), lambda i: (i, 0)),
            pl.BlockSpec((IN_F, F1), lambda i: (0, 0)),
            pl.BlockSpec((1, F1), lambda i: (0, 0)),
            pl.BlockSpec((F1, HEADS), lambda i: (0, 0)),
            pl.BlockSpec((F1, HEADS), lambda i: (0, 0)),
        ],
        out_specs=[
            pl.BlockSpec((BLK, F1), lambda i: (i, 0)),
            pl.BlockSpec((HEADS, BLK), lambda i: (0, i)),
            pl.BlockSpec((HEADS, BLK), lambda i: (0, i)),
        ],
        out_shape=[
            jax.ShapeDtypeStruct((NP, F1), jnp.float32),
            jax.ShapeDtypeStruct((HEADS, NP), jnp.float32),
            jax.ShapeDtypeStruct((HEADS, NP), jnp.float32),
        ],
    )(xp, wcatT, bcat, a1, a2)


def _comb_body(d_ref, x_ref, erep_ref, xs_ref):
    dsum = d_ref[0] + d_ref[1]
    dinv = 1.0 / dsum
    mult = lax.dot_general(dinv, erep_ref[...], (((0,), (0,)), ((), ())),
                           preferred_element_type=jnp.float32)
    xs_ref[...] = x_ref[...] * mult


def _comb(denom, xarr, erep):
    nh, f = erep.shape
    return pl.pallas_call(
        _comb_body,
        grid=(GRID,),
        in_specs=[
            pl.BlockSpec((2, nh, BLK), lambda i: (0, 0, i)),
            pl.BlockSpec((BLK, f), lambda i: (i, 0)),
            pl.BlockSpec((nh, f), lambda i: (0, 0)),
        ],
        out_specs=pl.BlockSpec((BLK, f), lambda i: (i, 0)),
        out_shape=jax.ShapeDtypeStruct((NP, f), jnp.float32),
    )(denom, xarr, erep)


def _lin2_body(p_ref, wt_ref, b_ref, a1_ref, a2_ref, x2_ref, s1t_ref, s2t_ref):
    i = pl.program_id(0)
    h = p_ref[0] + p_ref[1]
    gid = i * BLK + lax.broadcasted_iota(jnp.int32, (BLK, 1), 0)
    h = jnp.where(gid < N, h, 0.0)
    x2 = jnp.dot(h, wt_ref[...], preferred_element_type=jnp.float32)
    x2 = x2 + b_ref[...]
    x2_ref[...] = x2
    dn = (((0,), (1,)), ((), ()))
    s1t_ref[...] = lax.dot_general(a1_ref[...], x2, dn,
                                   preferred_element_type=jnp.float32)
    s2t_ref[...] = lax.dot_general(a2_ref[...], x2, dn,
                                   preferred_element_type=jnp.float32)


def _lin2(outp1, woT, bo2, a1o, a2o):
    return pl.pallas_call(
        _lin2_body,
        grid=(GRID,),
        in_specs=[
            pl.BlockSpec((2, BLK, F1), lambda i: (0, i, 0)),
            pl.BlockSpec((F1, F1), lambda i: (0, 0)),
            pl.BlockSpec((1, F1), lambda i: (0, 0)),
            pl.BlockSpec((F1, 1), lambda i: (0, 0)),
            pl.BlockSpec((F1, 1), lambda i: (0, 0)),
        ],
        out_specs=[
            pl.BlockSpec((BLK, F1), lambda i: (i, 0)),
            pl.BlockSpec((1, BLK), lambda i: (0, i)),
            pl.BlockSpec((1, BLK), lambda i: (0, i)),
        ],
        out_shape=[
            jax.ShapeDtypeStruct((NP, F1), jnp.float32),
            jax.ShapeDtypeStruct((1, NP), jnp.float32),
            jax.ShapeDtypeStruct((1, NP), jnp.float32),
        ],
    )(outp1, woT, bo2, a1o, a2o)


def _final_body(p_ref, o_ref):
    s = p_ref[0] + p_ref[1]
    s = s[:, :F2]
    e = jnp.where(s > 0, s, jnp.exp(s) - 1.0)
    m = jnp.max(e, axis=1, keepdims=True)
    z = jnp.exp(e - m)
    o_ref[...] = z / jnp.sum(z, axis=1, keepdims=True)


def _final(outp2):
    return pl.pallas_call(
        _final_body,
        grid=(GRID,),
        in_specs=[pl.BlockSpec((2, BLK, F1), lambda i: (0, i, 0))],
        out_specs=pl.BlockSpec((BLK, F2), lambda i: (i, 0)),
        out_shape=jax.ShapeDtypeStruct((NP, F2), jnp.float32),
    )(outp2)


_MESH = plsc.VectorSubcoreMesh(core_axis_name="c", subcore_axis_name="s",
                               num_cores=NC, num_subcores=NS)


def _make_edge_w(nh):
    @functools.partial(
        pl.kernel,
        out_type=(
            jax.ShapeDtypeStruct((NC, nh, NP), jnp.float32),
            jax.ShapeDtypeStruct((NW, NCHA, CK * nh), jnp.float32),
        ),
        mesh=_MESH,
        scratch_types=[
            pltpu.VMEM((nh * NP,), jnp.float32),
            pltpu.VMEM((nh * NP,), jnp.float32),
            pltpu.VMEM((NCHA, CK), jnp.int32),
            pltpu.VMEM((NCHA, CK), jnp.int32),
            pltpu.VMEM((nh, CK), jnp.float32),
            pltpu.VMEM((CK * nh,), jnp.float32),
            [pltpu.VMEM_SHARED((NP,), jnp.float32)] * nh,
        ],
        compiler_params=pltpu.CompilerParams(needs_layout_passes=False),
    )
    def k(s1_hbm, s2_hbm, row_hbm, col_hbm, denom_hbm, w_hbm,
          s1_v, s2_v, row_v, col_v, wt_v, w_v, accs):
        cid = lax.axis_index("c")
        sid = lax.axis_index("s")
        wid = sid * NC + cid
        zf = jnp.zeros((16,), jnp.float32)
        for h in range(nh):
            for g in range(8):
                wt_v[h, pl.ds(g * 16, 16)] = zf
        for h in range(nh):
            for t in range(RPT // CK):
                pltpu.sync_copy(
                    wt_v.at[h],
                    accs[h].at[pl.ds(sid * RPT + t * CK, CK)])
        plsc.subcore_barrier()
        for h in range(nh):
            pltpu.sync_copy(s1_hbm.at[h], s1_v.at[pl.ds(h * NP, NP)])
            pltpu.sync_copy(s2_hbm.at[h], s2_v.at[pl.ds(h * NP, NP)])
        pltpu.sync_copy(row_hbm.at[wid], row_v)
        pltpu.sync_copy(col_hbm.at[wid], col_v)

        def chunk(j, carry):
            for g in range(8):
                kv = g * 16 + lax.iota(jnp.int32, 16)
                ridx = row_v[j, pl.ds(g * 16, 16)]
                cidx = col_v[j, pl.ds(g * 16, 16)]
                for h in range(nh):
                    s1 = plsc.load_gather(s1_v, [ridx + (h * NP)])
                    s2 = plsc.load_gather(s2_v, [cidx + (h * NP)])
                    t = s1 + s2
                    e = jnp.where(t > 0, t, t * 0.01)
                    w = jnp.exp(e)
                    wt_v[h, pl.ds(g * 16, 16)] = w
                    plsc.store_scatter(w_v, [kv * nh + h], w)
            for h in range(nh):
                pltpu.sync_copy(wt_v.at[h], accs[h].at[row_v.at[j]],
                                add=True)
            pltpu.sync_copy(w_v, w_hbm.at[wid, j])
            return carry

        lax.fori_loop(0, NCHA, chunk, 0)
        plsc.subcore_barrier()
        for h in range(nh):
            pltpu.sync_copy(accs[h].at[pl.ds(sid * RPT, RPT)],
                            denom_hbm.at[cid, h, pl.ds(sid * RPT, RPT)])

    return k


def _make_edge_agg(nh, F):
    nblk = F // 16
    fph = F // nh

    @functools.partial(
        pl.kernel,
        out_type=jax.ShapeDtypeStruct((NC, NP, F), jnp.float32),
        mesh=_MESH,
        scratch_types=[
            [pltpu.VMEM((1, CKB), jnp.int32)] * 2,
            [pltpu.VMEM((1, CKB), jnp.int32)] * 2,
            [pltpu.VMEM((1, CKB * nh + 16), jnp.float32)] * 2,
            [pltpu.VMEM((CKB, F), jnp.float32)] * 2,
            [pltpu.SemaphoreType.DMA] * 2,
            [pltpu.SemaphoreType.DMA] * 2,
            [pltpu.SemaphoreType.DMA] * 2,
            [pltpu.SemaphoreType.DMA] * 2,
            pltpu.VMEM_SHARED((NP, F), jnp.float32),
        ],
        compiler_params=pltpu.CompilerParams(needs_layout_passes=False),
    )
    def k(xs_hbm, row_hbm, col_hbm, w_hbm, out_hbm,
          ribufs, cibufs, wbufs, msgs, risems, cisems, gsems, wsems,
          acc_sh):
        cid = lax.axis_index("c")
        sid = lax.axis_index("s")
        wid = sid * NC + cid
        zf = jnp.zeros((16,), jnp.float32)

        def zrow(i, carry):
            for b in range(nblk):
                msgs[0][i, pl.ds(b * 16, 16)] = zf
            return carry

        lax.fori_loop(0, CKB, zrow, 0)
        for t in range(RPT // CKB):
            pltpu.sync_copy(msgs[0],
                            acc_sh.at[pl.ds(sid * RPT + t * CKB, CKB)])
        plsc.subcore_barrier()

        def ristart(j, b):
            pltpu.async_copy(row_hbm.at[wid, j], ribufs[b], risems[b])

        def riwait(j, b):
            pltpu.make_async_copy(row_hbm.at[wid, j], ribufs[b],
                                  risems[b]).wait()

        def cistart(j, b):
            pltpu.async_copy(col_hbm.at[wid, j], cibufs[b], cisems[b])

        def ciwait(j, b):
            pltpu.make_async_copy(col_hbm.at[wid, j], cibufs[b],
                                  cisems[b]).wait()

        def gstart(j, b):
            pltpu.async_copy(w_hbm.at[wid, j],
                             wbufs[b].at[:, pl.ds(0, CKB * nh)], wsems[b])
            pltpu.async_copy(xs_hbm.at[ribufs[b].at[0]], msgs[b], gsems[b])

        def gwait(j, b):
            pltpu.make_async_copy(w_hbm.at[wid, j],
                                  wbufs[b].at[:, pl.ds(0, CKB * nh)],
                                  wsems[b]).wait()
            pltpu.make_async_copy(xs_hbm.at[ribufs[b].at[0]], msgs[b],
                                  gsems[b]).wait()

        def compute(j, b):
            mb = msgs[b]
            wb = wbufs[b]

            def edge(kk, c2):
                wvec = wb[0, pl.ds(kk * nh, 16)]
                for bb in range(nblk):
                    h = (bb * 16) // fph
                    v = mb[kk, pl.ds(bb * 16, 16)]
                    mb[kk, pl.ds(bb * 16, 16)] = v * wvec[h]
                return c2

            lax.fori_loop(0, CKB, edge, 0, unroll=4)

        pltpu.sync_copy(row_hbm.at[wid, 0], ribufs[0])
        pltpu.sync_copy(col_hbm.at[wid, 0], cibufs[0])
        ristart(1, 1)
        cistart(1, 1)
        gstart(0, 0)
        half = NCHB // 2

        def outer(i, carry):
            for b in range(2):
                j = i * 2 + b
                riwait(j + 1, 1 - b)
                gstart(j + 1, 1 - b)
                gwait(j, b)

                @pl.when(j + 2 < NCHB)
                def _():
                    ristart(j + 2, b)
                compute(j, b)
                if b == 0:
                    @pl.when(i > 0)
                    def _():
                        ciwait(j, b)
                else:
                    ciwait(j, b)
                pltpu.sync_copy(msgs[b], acc_sh.at[cibufs[b].at[0]],
                                add=True)

                @pl.when(j + 2 < NCHB)
                def _():
                    cistart(j + 2, b)
            return carry

        lax.fori_loop(0, half, outer, 0)
        jl = NCHB - 1
        gwait(jl, 0)
        compute(jl, 0)
        ciwait(jl, 0)
        pltpu.sync_copy(msgs[0], acc_sh.at[cibufs[0].at[0]], add=True)
        plsc.subcore_barrier()
        pltpu.sync_copy(acc_sh.at[pl.ds(sid * RPT, RPT)],
                        out_hbm.at[cid, pl.ds(sid * RPT, RPT)])

    return k


_edge_w4 = _make_edge_w(4)
_edge_w1 = _make_edge_w(1)
_edge_agg4 = _make_edge_agg(4, F1)
_edge_agg1 = _make_edge_agg(1, F1)


def kernel(x, edge_index, dropout, Wh, bh, ah, Wo, bo, ao):
    f32 = jnp.float32
    x = x.astype(f32)
    row = edge_index[0].astype(jnp.int32)
    col = edge_index[1].astype(jnp.int32)

    xp = jnp.pad(x, ((0, NPAD), (0, 0)))
    wcatT = Wh.reshape(F1, IN_F).T
    bcat = bh.reshape(1, F1)
    eye = jnp.eye(HEADS, dtype=f32)
    a1 = (eye[:, None, :] * ah[:, :HID, 0][:, :, None]).reshape(F1, HEADS)
    a2 = (eye[:, None, :] * ah[:, HID:, 0][:, :, None]).reshape(F1, HEADS)
    erep = jnp.repeat(eye, HID, axis=1)
    erep2 = jnp.ones((1, F1), f32)
    woT = jnp.pad(Wo.T, ((0, 0), (0, F1 - F2)))
    bo2 = jnp.pad(bo.reshape(1, F2), ((0, 0), (0, F1 - F2)))
    a1o = jnp.pad(ao[:F2], ((0, F1 - F2), (0, 0)))
    a2o = jnp.pad(ao[F2:], ((0, F1 - F2), (0, 0)))

    dummy = jnp.broadcast_to(
        N + jnp.arange(EWP - EW, dtype=jnp.int32) % NPAD, (NW, EWP - EW))
    rowf = jnp.concatenate([row.reshape(NW, EW), dummy], axis=1)
    colf = jnp.concatenate([col.reshape(NW, EW), dummy], axis=1)
    rowp = rowf.reshape(NW, NCHA, CK)
    colp = colf.reshape(NW, NCHA, CK)
    rowb = rowf.reshape(NW, NCHB, 1, CKB)
    colb = colf.reshape(NW, NCHB, 1, CKB)

    x1, s1t, s2t = _lin1(xp, wcatT, bcat, a1, a2)
    denom1, w1 = _edge_w4(s1t, s2t, rowp, colp)
    xs1 = _comb(denom1, x1, erep)
    outp1 = _edge_agg4(xs1, rowb, colb,
                       w1.reshape(NW, NCHB, 1, CKB * HEADS))

    x2, s1ot, s2ot = _lin2(outp1, woT, bo2, a1o, a2o)
    denom2, w2 = _edge_w1(s1ot, s2ot, rowp, colp)
    xs2 = _comb(denom2, x2, erep2)
    outp2 = _edge_agg1(xs2, rowb, colb, w2.reshape(NW, NCHB, 1, CKB))

    out = _final(outp2)
    return out[:N]
